# baseline (speedup 1.0000x reference)
import functools

import jax
import jax.numpy as jnp
from jax.experimental import pallas as pl
from jax.experimental.pallas import tpu as pltpu


def _prep_body(x_ref, w_ref, as_ref, an_ref, xp_ref, s_ref, n_ref, m_ref):
    xp = jnp.dot(x_ref[...], w_ref[...], preferred_element_type=jnp.float32)
    xp_ref[...] = xp
    s = jnp.dot(xp, as_ref[...], preferred_element_type=jnp.float32)
    n = jnp.dot(xp, an_ref[...], preferred_element_type=jnp.float32)
    s_ref[...] = s
    n_ref[...] = n
    e = s + jnp.max(n, axis=0, keepdims=True)
    m_ref[...] = jnp.maximum(jnp.maximum(e, 0.2 * e), 0.0)


def _prep(x, wflat, a_s, a_n):
    n_nodes = x.shape[0]
    hc = wflat.shape[1]
    h = a_s.shape[1]
    f32 = jnp.float32
    return pl.pallas_call(
        _prep_body,
        out_shape=[
            jax.ShapeDtypeStruct((n_nodes, hc), f32),
            jax.ShapeDtypeStruct((n_nodes, h), f32),
            jax.ShapeDtypeStruct((n_nodes, h), f32),
            jax.ShapeDtypeStruct((n_nodes, h), f32),
        ],
    )(x, wflat, a_s, a_n)


def _agg_body(a_ref, s_ref, m_ref, n_ref, xp_ref, b_ref, out_ref, acc_ref, z_ref,
              *, heads, chan, n_j_blocks, act):
    j = pl.program_id(1)

    @pl.when(j == 0)
    def _():
        acc_ref[...] = jnp.zeros_like(acc_ref)
        z_ref[...] = jnp.zeros_like(z_ref)

    a_blk = a_ref[...]
    for h in range(heads):
        e = s_ref[:, h:h + 1] + n_ref[:, h].reshape(1, -1)
        e = jnp.maximum(e, 0.2 * e)
        p = a_blk * jnp.exp(e - m_ref[:, h:h + 1])
        z_ref[:, h:h + 1] += jnp.sum(p, axis=1, keepdims=True)
        acc_ref[:, h * chan:(h + 1) * chan] += jnp.dot(
            p, xp_ref[:, h * chan:(h + 1) * chan],
            preferred_element_type=jnp.float32)

    @pl.when(j == n_j_blocks - 1)
    def _():
        acc = acc_ref[...]
        parts = [acc[:, h * chan:(h + 1) * chan] / z_ref[:, h:h + 1]
                 for h in range(heads)]
        t = jnp.concatenate(parts, axis=1) if len(parts) > 1 else parts[0]
        t = t + b_ref[...]
        out_ref[...] = act(t)


def _elu(t):
    return jnp.where(t > 0, t, jnp.exp(t) - 1.0)


def _row_softmax(t):
    m = jnp.max(t, axis=1, keepdims=True)
    ex = jnp.exp(t - m)
    return ex / jnp.sum(ex, axis=1, keepdims=True)


def _agg(adj, xp, s, n, m, brow, heads, chan, act, rb=256, cb=512):
    n_nodes = adj.shape[0]
    hc = xp.shape[1]
    out_c = heads * chan
    grid = (n_nodes // rb, n_nodes // cb)
    body = functools.partial(_agg_body, heads=heads, chan=chan,
                             n_j_blocks=grid[1], act=act)
    return pl.pallas_call(
        body,
        grid=grid,
        in_specs=[
            pl.BlockSpec((rb, cb), lambda i, j: (i, j)),
            pl.BlockSpec((rb, heads), lambda i, j: (i, 0)),
            pl.BlockSpec((rb, heads), lambda i, j: (i, 0)),
            pl.BlockSpec((cb, heads), lambda i, j: (j, 0)),
            pl.BlockSpec((cb, hc), lambda i, j: (j, 0)),
            pl.BlockSpec((1, out_c), lambda i, j: (0, 0)),
        ],
        out_specs=pl.BlockSpec((rb, out_c), lambda i, j: (i, 0)),
        out_shape=jax.ShapeDtypeStruct((n_nodes, out_c), jnp.float32),
        scratch_shapes=[
            pltpu.VMEM((rb, out_c), jnp.float32),
            pltpu.VMEM((rb, heads), jnp.float32),
        ],
        compiler_params=pltpu.CompilerParams(
            dimension_semantics=("parallel", "arbitrary")),
    )(adj, s, m, n, xp, brow)


def _head_proj(a_vec, heads):
    return jnp.einsum('ch,hk->hck', a_vec, jnp.eye(heads, dtype=a_vec.dtype)
                      ).reshape(-1, heads)


def kernel(x, fltr, W1, a1_self, a1_neigh, b1, W2, a2_self, a2_neigh, b2):
    f, h1, c1 = W1.shape
    hc1, h2, c2 = W2.shape

    w1flat = W1.reshape(f, h1 * c1)
    xp1, s1, n1, m1 = _prep(x, w1flat, _head_proj(a1_self, h1),
                            _head_proj(a1_neigh, h1))
    hid = _agg(fltr, xp1, s1, n1, m1, b1.reshape(1, -1), h1, c1, _elu)

    w2flat = W2.reshape(hc1, h2 * c2)
    xp2, s2, n2, m2 = _prep(hid, w2flat, _head_proj(a2_self, h2),
                            _head_proj(a2_neigh, h2))
    out = _agg(fltr, xp2, s2, n2, m2, b2.reshape(1, -1), h2, c2, _row_softmax)
    return out
